# baseline (speedup 1.0000x reference)
import numpy as np
import jax
import jax.numpy as jnp
from jax.experimental import pallas as pl
from jax.experimental.pallas import tpu as pltpu


_VMEM_LIMIT = 48 * 1024 * 1024


def _tap_offsets(dp, p):
    return tuple(p + (kd - 1) * dp * dp + (kh - 1) * dp + (kw - 1)
                 for kd in range(3) for kh in range(3) for kw in range(3))


def _halo_mask(dp):
    m = np.zeros((dp, dp, dp), np.float32)
    m[1:-1, 1:-1, 1:-1] = 1.0
    return jnp.asarray(m.reshape(dp * dp * dp, 1))


def _pick_chunk(sp, cap=2048):
    if sp <= cap:
        return sp
    for n in range(2, sp + 1):
        if sp % n == 0 and sp // n <= cap:
            return sp // n
    return sp


def _conv_stage(src_ref, w_ref, scale_ref, shift_ref, mask_ref, dst_ref, im_ref,
                *, sp, p, offsets, chunk, cin, relu):
    cout = dst_ref.shape[-1]
    zpad = jnp.zeros((p, cout), dst_ref.dtype)
    dst_ref[0:p, :] = zpad
    dst_ref[p + sp:p + sp + p, :] = zpad
    for base in range(0, sp, chunk):
        for t, off in enumerate(offsets):
            im_ref[:, t * cin:(t + 1) * cin] = \
                src_ref[off + base:off + base + chunk, :]
        acc = jnp.dot(im_ref[...], w_ref[...],
                      preferred_element_type=jnp.float32)
        res = acc * scale_ref[...] + shift_ref[...]
        if relu:
            res = jnp.maximum(res, 0.0)
        res = res * mask_ref[base:base + chunk, :]
        dst_ref[p + base:p + base + chunk, :] = res.astype(dst_ref.dtype)


def _make_c12_body(sp, p, offsets, chunk, c1_out):
    def _body(x27_ref, w1_ref, w2_ref, s2_ref, b2_ref, mask_ref, o_ref,
              mid_ref, im_ref):
        zpad = jnp.zeros((p, c1_out), mid_ref.dtype)
        mid_ref[0:p, :] = zpad
        mid_ref[p + sp:p + sp + p, :] = zpad
        for base in range(0, sp, chunk):
            cols = x27_ref[base:base + chunk, :]
            acc = jnp.dot(cols, w1_ref[...], preferred_element_type=jnp.float32)
            nrm = jnp.sqrt(jnp.sum(acc * acc, axis=-1, keepdims=True)) + 1e-9
            res = jnp.maximum(acc / nrm, 0.0) * mask_ref[base:base + chunk, :]
            mid_ref[p + base:p + base + chunk, :] = res.astype(mid_ref.dtype)
        _conv_stage(mid_ref, w2_ref, s2_ref, b2_ref, mask_ref, o_ref, im_ref,
                    sp=sp, p=p, offsets=offsets, chunk=chunk, cin=c1_out,
                    relu=False)
    return _body


def _conv12(x27, w1, w2, s2, b2, D):
    B = x27.shape[0]
    Dp = D + 2
    Sp = Dp ** 3
    P = Dp * Dp + Dp + 1
    chunk = _pick_chunk(Sp)
    offsets = _tap_offsets(Dp, P)
    mask = _halo_mask(Dp)
    c1_out = w1.shape[-1]
    c2_out = w2.shape[-1]
    return pl.pallas_call(
        _make_c12_body(Sp, P, offsets, chunk, c1_out),
        out_shape=jax.ShapeDtypeStruct((B, Sp + 2 * P, c2_out), jnp.bfloat16),
        grid=(B,),
        in_specs=[
            pl.BlockSpec((None, Sp, 27), lambda b: (b, 0, 0)),
            pl.BlockSpec((27, c1_out), lambda b: (0, 0)),
            pl.BlockSpec((27 * c1_out, c2_out), lambda b: (0, 0)),
            pl.BlockSpec((1, c2_out), lambda b: (0, 0)),
            pl.BlockSpec((1, c2_out), lambda b: (0, 0)),
            pl.BlockSpec((Sp, 1), lambda b: (0, 0)),
        ],
        out_specs=pl.BlockSpec((None, Sp + 2 * P, c2_out), lambda b: (b, 0, 0)),
        scratch_shapes=[
            pltpu.VMEM((Sp + 2 * P, c1_out), jnp.bfloat16),
            pltpu.VMEM((chunk, 27 * c1_out), jnp.bfloat16),
        ],
        compiler_params=pltpu.CompilerParams(
            dimension_semantics=("parallel",),
            vmem_limit_bytes=_VMEM_LIMIT),
    )(x27, w1, w2, s2, b2, mask)


def _make_pair_body(sp, p, offsets, chunk, cin_a, cin_b):
    def _body(x_ref, wa_ref, sa_ref, ba_ref, wb_ref, sb_ref, bb_ref,
              mask_ref, o_ref, mid_ref, im_a_ref, im_b_ref):
        _conv_stage(x_ref, wa_ref, sa_ref, ba_ref, mask_ref, mid_ref, im_a_ref,
                    sp=sp, p=p, offsets=offsets, chunk=chunk, cin=cin_a,
                    relu=True)
        _conv_stage(mid_ref, wb_ref, sb_ref, bb_ref, mask_ref, o_ref, im_b_ref,
                    sp=sp, p=p, offsets=offsets, chunk=chunk, cin=cin_b,
                    relu=True)
    return _body


def _conv_pair(x_flat, pa, pb, D):
    B, total, cin = x_flat.shape
    Dp = D + 2
    Sp = Dp ** 3
    P = Dp * Dp + Dp + 1
    assert total == Sp + 2 * P
    wa, sa, ba = pa
    wb, sb, bb = pb
    ca_out = wa.shape[-1]
    cb_out = wb.shape[-1]
    chunk = _pick_chunk(Sp)
    offsets = _tap_offsets(Dp, P)
    mask = _halo_mask(Dp)
    return pl.pallas_call(
        _make_pair_body(Sp, P, offsets, chunk, cin, ca_out),
        out_shape=jax.ShapeDtypeStruct((B, Sp + 2 * P, cb_out), jnp.bfloat16),
        grid=(B,),
        in_specs=[
            pl.BlockSpec((None, Sp + 2 * P, cin), lambda b: (b, 0, 0)),
            pl.BlockSpec((27 * cin, ca_out), lambda b: (0, 0)),
            pl.BlockSpec((1, ca_out), lambda b: (0, 0)),
            pl.BlockSpec((1, ca_out), lambda b: (0, 0)),
            pl.BlockSpec((27 * ca_out, cb_out), lambda b: (0, 0)),
            pl.BlockSpec((1, cb_out), lambda b: (0, 0)),
            pl.BlockSpec((1, cb_out), lambda b: (0, 0)),
            pl.BlockSpec((Sp, 1), lambda b: (0, 0)),
        ],
        out_specs=pl.BlockSpec((None, Sp + 2 * P, cb_out), lambda b: (b, 0, 0)),
        scratch_shapes=[
            pltpu.VMEM((Sp + 2 * P, ca_out), jnp.bfloat16),
            pltpu.VMEM((chunk, 27 * cin), jnp.bfloat16),
            pltpu.VMEM((chunk, 27 * ca_out), jnp.bfloat16),
        ],
        compiler_params=pltpu.CompilerParams(
            dimension_semantics=("parallel",),
            vmem_limit_bytes=_VMEM_LIMIT),
    )(x_flat, wa, sa, ba, wb, sb, bb, mask)


def _head_body(x_ref, w9_ref, s9_ref, b9_ref, w10_ref, s10_ref, b10_ref,
               w11_ref, s11_ref, b11_ref, o_ref):
    h = jnp.dot(x_ref[...], w9_ref[...], preferred_element_type=jnp.float32)
    h = jnp.maximum(h * s9_ref[...] + b9_ref[...], 0.0)
    h = jnp.dot(h.astype(jnp.bfloat16), w10_ref[...],
                preferred_element_type=jnp.float32)
    h = jnp.maximum(h * s10_ref[...] + b10_ref[...], 0.0)
    h = jnp.dot(h.astype(jnp.bfloat16), w11_ref[...],
                preferred_element_type=jnp.float32)
    h = jnp.maximum(h * s11_ref[...] + b11_ref[...], 0.0)
    z = h - jnp.max(h, axis=1, keepdims=True)
    e = jnp.exp(z)
    o_ref[...] = e / jnp.sum(e, axis=1, keepdims=True)


def _head(v, head_params):
    B, C = v.shape
    (w9, s9, b9), (w10, s10, b10), (w11, s11, b11) = head_params
    nc = w11.shape[-1]
    return pl.pallas_call(
        _head_body,
        out_shape=jax.ShapeDtypeStruct((B, nc), jnp.float32),
        in_specs=[pl.BlockSpec(v.shape, lambda: (0, 0))] +
                 [pl.BlockSpec(a.shape, lambda: (0, 0))
                  for a in (w9, s9, b9, w10, s10, b10, w11, s11, b11)],
        out_specs=pl.BlockSpec((B, nc), lambda: (0, 0)),
        compiler_params=pltpu.CompilerParams(
            vmem_limit_bytes=_VMEM_LIMIT),
    )(v, w9, s9, b9, w10, s10, b10, w11, s11, b11)


def _unpad_flat(h_flat, D, C):
    B = h_flat.shape[0]
    Dp = D + 2
    Sp = Dp ** 3
    P = Dp * Dp + Dp + 1
    v = h_flat[:, P:P + Sp, :].reshape(B, Dp, Dp, Dp, C)
    return v[:, 1:1 + D, 1:1 + D, 1:1 + D, :]


def _maxpool2(v):
    B, D = v.shape[0], v.shape[1]
    C = v.shape[-1]
    Do = D // 2
    return v.reshape(B, Do, 2, Do, 2, Do, 2, C).max(axis=(2, 4, 6))


def _pad_flat(v):
    B, D = v.shape[0], v.shape[1]
    C = v.shape[-1]
    Dp = D + 2
    P = Dp * Dp + Dp + 1
    vp = jnp.pad(v, ((0, 0), (1, 1), (1, 1), (1, 1), (0, 0)))
    return jnp.pad(vp.reshape(B, Dp ** 3, C), ((0, 0), (P, P), (0, 0)))


@jax.jit
def _forward(x, params, head_params):
    B, D = x.shape[0], x.shape[1]
    xb = x.astype(jnp.bfloat16)
    Dp = D + 2
    Sp = Dp ** 3
    xp2 = jnp.pad(xb, ((0, 0), (2, 2), (2, 2), (2, 2)))
    cols = [xp2[:, kd:kd + Dp, kh:kh + Dp, kw:kw + Dp].reshape(B, Sp)
            for kd in range(3) for kh in range(3) for kw in range(3)]
    x27 = jnp.stack(cols, axis=-1)

    h = _conv12(x27, params[0][0], params[1][0], params[1][1], params[1][2], D)
    v = _maxpool2(_unpad_flat(h, D, 32))
    h = _conv_pair(_pad_flat(v), params[2], params[3], D // 2)
    v = _maxpool2(_unpad_flat(h, D // 2, 64))
    h = _conv_pair(_pad_flat(v), params[4], params[5], D // 4)
    v = _maxpool2(_unpad_flat(h, D // 4, 128))
    h = _conv_pair(_pad_flat(v), params[6], params[7], D // 8)
    v = _maxpool2(_unpad_flat(h, D // 8, 256))
    return _head(v.reshape(B, 256), head_params)


def kernel(x, w0, s0, sh0, w1, s1, sh1, w2, s2, sh2, w3, s3, sh3,
           w4, s4, sh4, w5, s5, sh5, w6, s6, sh6, w7, s7, sh7,
           w8, s8, sh8, w9, s9, sh9, w10, s10, sh10,
           hw0, hs0, hb0, hw1, hs1, hb1, hw2, hs2, hb2):
    params = [(w0, s0, sh0), (w1, s1, sh1), (w2, s2, sh2), (w3, s3, sh3),
              (w4, s4, sh4), (w5, s5, sh5), (w6, s6, sh6), (w7, s7, sh7)]
    head_params = ((hw0, hs0, hb0), (hw1, hs1, hb1), (hw2, hs2, hb2))
    return _forward(x, params, head_params)


# speedup vs baseline: 1.1518x; 1.1518x over previous
import numpy as np
import jax
import jax.numpy as jnp
from jax.experimental import pallas as pl
from jax.experimental.pallas import tpu as pltpu


_VMEM_LIMIT = 48 * 1024 * 1024


def _tap_offsets(dp, p):
    return tuple(p + (kd - 1) * dp * dp + (kh - 1) * dp + (kw - 1)
                 for kd in range(3) for kh in range(3) for kw in range(3))


def _halo_mask(dp):
    m = np.zeros((dp, dp, dp), np.float32)
    m[1:-1, 1:-1, 1:-1] = 1.0
    return jnp.asarray(m.reshape(dp * dp * dp, 1))


def _pick_chunk(sp, cap=2048):
    if sp <= cap:
        return sp
    for n in range(2, sp + 1):
        if sp % n == 0 and sp // n <= cap:
            return sp // n
    return sp


def _conv_stage(src_ref, w_ref, scale_ref, shift_ref, mask_ref, dst_ref,
                im_refs, *, sp, p, dpp, offsets, chunk, cin, relu):
    cout = dst_ref.shape[-1]
    dst_ref[0:p + dpp, :] = jnp.zeros((p + dpp, cout), dst_ref.dtype)
    dst_ref[p + sp - dpp:p + sp + p, :] = \
        jnp.zeros((p + dpp, cout), dst_ref.dtype)
    for ci, base in enumerate(range(dpp, sp - dpp, chunk)):
        im_ref = im_refs[ci % len(im_refs)]
        for t, off in enumerate(offsets):
            im_ref[:, t * cin:(t + 1) * cin] = \
                src_ref[off + base:off + base + chunk, :]
        acc = jnp.dot(im_ref[...], w_ref[...],
                      preferred_element_type=jnp.float32)
        res = acc * scale_ref[...] + shift_ref[...]
        if relu:
            res = jnp.maximum(res, 0.0)
        res = res * mask_ref[base:base + chunk, :]
        dst_ref[p + base:p + base + chunk, :] = res.astype(dst_ref.dtype)


def _conv_stage_tapdot(src_ref, w_ref, scale_ref, shift_ref, mask_ref,
                       dst_ref, *, sp, p, dpp, offsets, chunk, cin, relu):
    cout = dst_ref.shape[-1]
    dst_ref[0:p + dpp, :] = jnp.zeros((p + dpp, cout), dst_ref.dtype)
    dst_ref[p + sp - dpp:p + sp + p, :] = \
        jnp.zeros((p + dpp, cout), dst_ref.dtype)
    for base in range(dpp, sp - dpp, chunk):
        acc = None
        for t, off in enumerate(offsets):
            part = jnp.dot(src_ref[off + base:off + base + chunk, :],
                           w_ref[t * cin:(t + 1) * cin, :],
                           preferred_element_type=jnp.float32)
            acc = part if acc is None else acc + part
        res = acc * scale_ref[...] + shift_ref[...]
        if relu:
            res = jnp.maximum(res, 0.0)
        res = res * mask_ref[base:base + chunk, :]
        dst_ref[p + base:p + base + chunk, :] = res.astype(dst_ref.dtype)


def _make_c12_body(sp, p, dpp, offsets, chunk, c1_out):
    def _body(x27_ref, w1_ref, w2_ref, s2_ref, b2_ref, mask_ref, o_ref,
              mid_ref, im_a_ref, im_b_ref):
        mid_ref[0:p + dpp, :] = jnp.zeros((p + dpp, c1_out), mid_ref.dtype)
        mid_ref[p + sp - dpp:p + sp + p, :] = \
            jnp.zeros((p + dpp, c1_out), mid_ref.dtype)
        for base in range(dpp, sp - dpp, chunk):
            cols = x27_ref[base:base + chunk, :]
            acc = jnp.dot(cols, w1_ref[...], preferred_element_type=jnp.float32)
            nrm = jnp.sqrt(jnp.sum(acc * acc, axis=-1, keepdims=True)) + 1e-9
            res = jnp.maximum(acc / nrm, 0.0) * mask_ref[base:base + chunk, :]
            mid_ref[p + base:p + base + chunk, :] = res.astype(mid_ref.dtype)
        _conv_stage(mid_ref, w2_ref, s2_ref, b2_ref, mask_ref, o_ref,
                    [im_a_ref, im_b_ref], sp=sp, p=p, dpp=dpp,
                    offsets=offsets, chunk=chunk, cin=c1_out, relu=False)
    return _body


def _conv12(x27, w1, w2, s2, b2, D):
    B = x27.shape[0]
    Dp = D + 2
    Sp = Dp ** 3
    P = Dp * Dp + Dp + 1
    Dpp = Dp * Dp
    chunk = _pick_chunk(Sp - 2 * Dpp)
    offsets = _tap_offsets(Dp, P)
    mask = _halo_mask(Dp)
    c1_out = w1.shape[-1]
    c2_out = w2.shape[-1]
    return pl.pallas_call(
        _make_c12_body(Sp, P, Dpp, offsets, chunk, c1_out),
        out_shape=jax.ShapeDtypeStruct((B, Sp + 2 * P, c2_out), jnp.bfloat16),
        grid=(B,),
        in_specs=[
            pl.BlockSpec((None, Sp, 27), lambda b: (b, 0, 0)),
            pl.BlockSpec((27, c1_out), lambda b: (0, 0)),
            pl.BlockSpec((27 * c1_out, c2_out), lambda b: (0, 0)),
            pl.BlockSpec((1, c2_out), lambda b: (0, 0)),
            pl.BlockSpec((1, c2_out), lambda b: (0, 0)),
            pl.BlockSpec((Sp, 1), lambda b: (0, 0)),
        ],
        out_specs=pl.BlockSpec((None, Sp + 2 * P, c2_out), lambda b: (b, 0, 0)),
        scratch_shapes=[
            pltpu.VMEM((Sp + 2 * P, c1_out), jnp.bfloat16),
            pltpu.VMEM((chunk, 27 * c1_out), jnp.bfloat16),
            pltpu.VMEM((chunk, 27 * c1_out), jnp.bfloat16),
        ],
        compiler_params=pltpu.CompilerParams(
            dimension_semantics=("parallel",),
            vmem_limit_bytes=_VMEM_LIMIT),
    )(x27, w1, w2, s2, b2, mask)


def _make_pair_body(sp, p, dpp, offsets, chunk, cin_a, cin_b):
    tap_b = cin_b >= 256

    def _body(x_ref, wa_ref, sa_ref, ba_ref, wb_ref, sb_ref, bb_ref,
              mask_ref, o_ref, mid_ref, *im_refs):
        _conv_stage(x_ref, wa_ref, sa_ref, ba_ref, mask_ref, mid_ref,
                    [im_refs[0]], sp=sp, p=p, dpp=dpp, offsets=offsets,
                    chunk=chunk, cin=cin_a, relu=True)
        if tap_b:
            _conv_stage_tapdot(mid_ref, wb_ref, sb_ref, bb_ref, mask_ref,
                               o_ref, sp=sp, p=p, dpp=dpp, offsets=offsets,
                               chunk=chunk, cin=cin_b, relu=True)
        else:
            _conv_stage(mid_ref, wb_ref, sb_ref, bb_ref, mask_ref, o_ref,
                        [im_refs[1]], sp=sp, p=p, dpp=dpp, offsets=offsets,
                        chunk=chunk, cin=cin_b, relu=True)
    return _body


def _conv_pair(x_flat, pa, pb, D):
    B, total, cin = x_flat.shape
    Dp = D + 2
    Sp = Dp ** 3
    P = Dp * Dp + Dp + 1
    assert total == Sp + 2 * P
    wa, sa, ba = pa
    wb, sb, bb = pb
    ca_out = wa.shape[-1]
    cb_out = wb.shape[-1]
    Dpp = Dp * Dp
    chunk = _pick_chunk(Sp - 2 * Dpp)
    offsets = _tap_offsets(Dp, P)
    mask = _halo_mask(Dp)
    scratch = [pltpu.VMEM((Sp + 2 * P, ca_out), jnp.bfloat16),
               pltpu.VMEM((chunk, 27 * cin), jnp.bfloat16)]
    if ca_out < 256:
        scratch.append(pltpu.VMEM((chunk, 27 * ca_out), jnp.bfloat16))
    return pl.pallas_call(
        _make_pair_body(Sp, P, Dpp, offsets, chunk, cin, ca_out),
        out_shape=jax.ShapeDtypeStruct((B, Sp + 2 * P, cb_out), jnp.bfloat16),
        grid=(B,),
        in_specs=[
            pl.BlockSpec((None, Sp + 2 * P, cin), lambda b: (b, 0, 0)),
            pl.BlockSpec((27 * cin, ca_out), lambda b: (0, 0)),
            pl.BlockSpec((1, ca_out), lambda b: (0, 0)),
            pl.BlockSpec((1, ca_out), lambda b: (0, 0)),
            pl.BlockSpec((27 * ca_out, cb_out), lambda b: (0, 0)),
            pl.BlockSpec((1, cb_out), lambda b: (0, 0)),
            pl.BlockSpec((1, cb_out), lambda b: (0, 0)),
            pl.BlockSpec((Sp, 1), lambda b: (0, 0)),
        ],
        out_specs=pl.BlockSpec((None, Sp + 2 * P, cb_out), lambda b: (b, 0, 0)),
        scratch_shapes=scratch,
        compiler_params=pltpu.CompilerParams(
            dimension_semantics=("parallel",),
            vmem_limit_bytes=_VMEM_LIMIT),
    )(x_flat, wa, sa, ba, wb, sb, bb, mask)


def _head_body(x_ref, w9_ref, s9_ref, b9_ref, w10_ref, s10_ref, b10_ref,
               w11_ref, s11_ref, b11_ref, o_ref):
    h = jnp.dot(x_ref[...], w9_ref[...], preferred_element_type=jnp.float32)
    h = jnp.maximum(h * s9_ref[...] + b9_ref[...], 0.0)
    h = jnp.dot(h.astype(jnp.bfloat16), w10_ref[...],
                preferred_element_type=jnp.float32)
    h = jnp.maximum(h * s10_ref[...] + b10_ref[...], 0.0)
    h = jnp.dot(h.astype(jnp.bfloat16), w11_ref[...],
                preferred_element_type=jnp.float32)
    h = jnp.maximum(h * s11_ref[...] + b11_ref[...], 0.0)
    z = h - jnp.max(h, axis=1, keepdims=True)
    e = jnp.exp(z)
    o_ref[...] = e / jnp.sum(e, axis=1, keepdims=True)


def _head(v, head_params):
    B, C = v.shape
    (w9, s9, b9), (w10, s10, b10), (w11, s11, b11) = head_params
    nc = w11.shape[-1]
    return pl.pallas_call(
        _head_body,
        out_shape=jax.ShapeDtypeStruct((B, nc), jnp.float32),
        in_specs=[pl.BlockSpec(v.shape, lambda: (0, 0))] +
                 [pl.BlockSpec(a.shape, lambda: (0, 0))
                  for a in (w9, s9, b9, w10, s10, b10, w11, s11, b11)],
        out_specs=pl.BlockSpec((B, nc), lambda: (0, 0)),
        compiler_params=pltpu.CompilerParams(
            vmem_limit_bytes=_VMEM_LIMIT),
    )(v, w9, s9, b9, w10, s10, b10, w11, s11, b11)


def _unpad_flat(h_flat, D, C):
    B = h_flat.shape[0]
    Dp = D + 2
    Sp = Dp ** 3
    P = Dp * Dp + Dp + 1
    v = h_flat[:, P:P + Sp, :].reshape(B, Dp, Dp, Dp, C)
    return v[:, 1:1 + D, 1:1 + D, 1:1 + D, :]


def _maxpool2(v):
    B, D = v.shape[0], v.shape[1]
    C = v.shape[-1]
    Do = D // 2
    return v.reshape(B, Do, 2, Do, 2, Do, 2, C).max(axis=(2, 4, 6))


def _pad_flat(v):
    B, D = v.shape[0], v.shape[1]
    C = v.shape[-1]
    Dp = D + 2
    P = Dp * Dp + Dp + 1
    vp = jnp.pad(v, ((0, 0), (1, 1), (1, 1), (1, 1), (0, 0)))
    return jnp.pad(vp.reshape(B, Dp ** 3, C), ((0, 0), (P, P), (0, 0)))


@jax.jit
def _forward(x, params, head_params):
    B, D = x.shape[0], x.shape[1]
    xb = x.astype(jnp.bfloat16)
    Dp = D + 2
    Sp = Dp ** 3
    xp2 = jnp.pad(xb, ((0, 0), (2, 2), (2, 2), (2, 2)))
    cols = [xp2[:, kd:kd + Dp, kh:kh + Dp, kw:kw + Dp].reshape(B, Sp)
            for kd in range(3) for kh in range(3) for kw in range(3)]
    x27 = jnp.stack(cols, axis=-1)

    h = _conv12(x27, params[0][0], params[1][0], params[1][1], params[1][2], D)
    v = _maxpool2(_unpad_flat(h, D, 32))
    h = _conv_pair(_pad_flat(v), params[2], params[3], D // 2)
    v = _maxpool2(_unpad_flat(h, D // 2, 64))
    h = _conv_pair(_pad_flat(v), params[4], params[5], D // 4)
    v = _maxpool2(_unpad_flat(h, D // 4, 128))
    h = _conv_pair(_pad_flat(v), params[6], params[7], D // 8)
    v = _maxpool2(_unpad_flat(h, D // 8, 256))
    return _head(v.reshape(B, 256), head_params)


def kernel(x, w0, s0, sh0, w1, s1, sh1, w2, s2, sh2, w3, s3, sh3,
           w4, s4, sh4, w5, s5, sh5, w6, s6, sh6, w7, s7, sh7,
           w8, s8, sh8, w9, s9, sh9, w10, s10, sh10,
           hw0, hs0, hb0, hw1, hs1, hb1, hw2, hs2, hb2):
    params = [(w0, s0, sh0), (w1, s1, sh1), (w2, s2, sh2), (w3, s3, sh3),
              (w4, s4, sh4), (w5, s5, sh5), (w6, s6, sh6), (w7, s7, sh7)]
    head_params = ((hw0, hs0, hb0), (hw1, hs1, hb1), (hw2, hs2, hb2))
    return _forward(x, params, head_params)


# speedup vs baseline: 2.5360x; 2.2017x over previous
import math

import numpy as np
import jax
import jax.numpy as jnp
from jax.experimental import pallas as pl
from jax.experimental.pallas import tpu as pltpu


_VMEM_LIMIT = 48 * 1024 * 1024


def _tap_offsets(dp, p):
    return tuple(p + (kd - 1) * dp * dp + (kh - 1) * dp + (kw - 1)
                 for kd in range(3) for kh in range(3) for kw in range(3))


def _halo_mask_t(dp):
    m = np.zeros((dp, dp, dp), np.float32)
    m[1:-1, 1:-1, 1:-1] = 1.0
    return jnp.asarray(m.reshape(1, dp * dp * dp))


def _pick_chunk(sp, cap=2048):
    if sp <= cap:
        return sp
    for n in range(2, sp + 1):
        if sp % n == 0 and sp // n <= cap:
            return sp // n
    return sp


def _tconv(srcs, w_ref, s_ref, b_ref, mask_ref, dsts, im_refs,
           *, sp, p, dpp, offsets, chunk, cin, relu):
    nb = len(srcs)
    cout = dsts[0].shape[0]
    zlead = jnp.zeros((cout, p + dpp), dsts[0].dtype)
    for dst in dsts:
        dst[:, 0:p + dpp] = zlead
        dst[:, p + sp - dpp:p + sp + p] = zlead
    for ci, base in enumerate(range(dpp, sp - dpp, chunk)):
        im_ref = im_refs[ci % len(im_refs)]
        for e, src in enumerate(srcs):
            for t, off in enumerate(offsets):
                im_ref[t * cin:(t + 1) * cin, e * chunk:(e + 1) * chunk] = \
                    src[:, off + base:off + base + chunk]
        acc = jnp.dot(w_ref[...], im_ref[...],
                      preferred_element_type=jnp.float32)
        res = acc * s_ref[...] + b_ref[...]
        if relu:
            res = jnp.maximum(res, 0.0)
        msk = mask_ref[:, base:base + chunk]
        for e, dst in enumerate(dsts):
            r = res[:, e * chunk:(e + 1) * chunk] * msk
            dst[:, p + base:p + base + chunk] = r.astype(dst.dtype)


def _make_c12_body(sp, p, dpp, offsets, chunk, c1_out):
    def _body(x27_ref, w1_ref, w2_ref, s2_ref, b2_ref, mask_ref, o_ref,
              mid_ref, im_a_ref, im_b_ref):
        mid_ref[:, 0:p + dpp] = jnp.zeros((c1_out, p + dpp), mid_ref.dtype)
        mid_ref[:, p + sp - dpp:p + sp + p] = \
            jnp.zeros((c1_out, p + dpp), mid_ref.dtype)
        for base in range(dpp, sp - dpp, chunk):
            acc = jnp.dot(w1_ref[...], x27_ref[:, base:base + chunk],
                          preferred_element_type=jnp.float32)
            nrm = jnp.sqrt(jnp.sum(acc * acc, axis=0, keepdims=True)) + 1e-9
            res = jnp.maximum(acc / nrm, 0.0) * mask_ref[:, base:base + chunk]
            mid_ref[:, p + base:p + base + chunk] = res.astype(mid_ref.dtype)
        _tconv([mid_ref], w2_ref, s2_ref, b2_ref, mask_ref, [o_ref.at[0]],
               [im_a_ref, im_b_ref], sp=sp, p=p, dpp=dpp, offsets=offsets,
               chunk=chunk, cin=c1_out, relu=False)
    return _body


def _conv12_t(x27, w1t, w2t, s2t, b2t, D):
    B = x27.shape[0]
    Dp = D + 2
    Sp = Dp ** 3
    P = Dp * Dp + Dp + 1
    Dpp = Dp * Dp
    chunk = _pick_chunk(Sp - 2 * Dpp)
    offsets = _tap_offsets(Dp, P)
    mask = _halo_mask_t(Dp)
    c1_out = w1t.shape[0]
    c2_out = w2t.shape[0]
    return pl.pallas_call(
        _make_c12_body(Sp, P, Dpp, offsets, chunk, c1_out),
        out_shape=jax.ShapeDtypeStruct((B, c2_out, Sp + 2 * P), jnp.bfloat16),
        grid=(B,),
        in_specs=[
            pl.BlockSpec((None, 27, Sp), lambda b: (b, 0, 0)),
            pl.BlockSpec((c1_out, 27), lambda b: (0, 0)),
            pl.BlockSpec((c2_out, 27 * c1_out), lambda b: (0, 0)),
            pl.BlockSpec((c2_out, 1), lambda b: (0, 0)),
            pl.BlockSpec((c2_out, 1), lambda b: (0, 0)),
            pl.BlockSpec((1, Sp), lambda b: (0, 0)),
        ],
        out_specs=pl.BlockSpec((1, c2_out, Sp + 2 * P), lambda b: (b, 0, 0)),
        scratch_shapes=[
            pltpu.VMEM((c1_out, Sp + 2 * P), jnp.bfloat16),
            pltpu.VMEM((27 * c1_out, chunk), jnp.bfloat16),
            pltpu.VMEM((27 * c1_out, chunk), jnp.bfloat16),
        ],
        compiler_params=pltpu.CompilerParams(
            dimension_semantics=("parallel",),
            vmem_limit_bytes=_VMEM_LIMIT),
    )(x27, w1t, w2t, s2t, b2t, mask)


def _make_pair_body(sp, p, dpp, offsets, chunk, nb, cin_a, cin_b):
    def _body(x_ref, wa_ref, sa_ref, ba_ref, wb_ref, sb_ref, bb_ref,
              mask_ref, o_ref, mid_ref, im_a_ref, im_b_ref):
        xs = [x_ref.at[e] for e in range(nb)]
        mids = [mid_ref.at[e] for e in range(nb)]
        outs = [o_ref.at[e] for e in range(nb)]
        _tconv(xs, wa_ref, sa_ref, ba_ref, mask_ref, mids, [im_a_ref],
               sp=sp, p=p, dpp=dpp, offsets=offsets, chunk=chunk,
               cin=cin_a, relu=True)
        _tconv(mids, wb_ref, sb_ref, bb_ref, mask_ref, outs, [im_b_ref],
               sp=sp, p=p, dpp=dpp, offsets=offsets, chunk=chunk,
               cin=cin_b, relu=True)
    return _body


def _conv_pair_t(x_t, pa, pb, D, nb):
    B, cin, total = x_t.shape
    Dp = D + 2
    Sp = Dp ** 3
    P = Dp * Dp + Dp + 1
    Dpp = Dp * Dp
    assert total == Sp + 2 * P and B % nb == 0
    wat, sat, bat = pa
    wbt, sbt, bbt = pb
    ca_out = wat.shape[0]
    cb_out = wbt.shape[0]
    chunk = Sp - 2 * Dpp
    offsets = _tap_offsets(Dp, P)
    mask = _halo_mask_t(Dp)
    return pl.pallas_call(
        _make_pair_body(Sp, P, Dpp, offsets, chunk, nb, cin, ca_out),
        out_shape=jax.ShapeDtypeStruct((B, cb_out, total), jnp.bfloat16),
        grid=(B // nb,),
        in_specs=[
            pl.BlockSpec((nb, cin, total), lambda b: (b, 0, 0)),
            pl.BlockSpec((ca_out, 27 * cin), lambda b: (0, 0)),
            pl.BlockSpec((ca_out, 1), lambda b: (0, 0)),
            pl.BlockSpec((ca_out, 1), lambda b: (0, 0)),
            pl.BlockSpec((cb_out, 27 * ca_out), lambda b: (0, 0)),
            pl.BlockSpec((cb_out, 1), lambda b: (0, 0)),
            pl.BlockSpec((cb_out, 1), lambda b: (0, 0)),
            pl.BlockSpec((1, Sp), lambda b: (0, 0)),
        ],
        out_specs=pl.BlockSpec((nb, cb_out, total), lambda b: (b, 0, 0)),
        scratch_shapes=[
            pltpu.VMEM((nb, ca_out, total), jnp.bfloat16),
            pltpu.VMEM((27 * cin, nb * chunk), jnp.bfloat16),
            pltpu.VMEM((27 * ca_out, nb * chunk), jnp.bfloat16),
        ],
        compiler_params=pltpu.CompilerParams(
            dimension_semantics=("parallel",),
            vmem_limit_bytes=_VMEM_LIMIT),
    )(x_t, wat, sat, bat, wbt, sbt, bbt, mask)


def _head_body(x_ref, w9_ref, s9_ref, b9_ref, w10_ref, s10_ref, b10_ref,
               w11_ref, s11_ref, b11_ref, o_ref):
    h = jnp.dot(x_ref[...], w9_ref[...], preferred_element_type=jnp.float32)
    h = jnp.maximum(h * s9_ref[...] + b9_ref[...], 0.0)
    h = jnp.dot(h.astype(jnp.bfloat16), w10_ref[...],
                preferred_element_type=jnp.float32)
    h = jnp.maximum(h * s10_ref[...] + b10_ref[...], 0.0)
    h = jnp.dot(h.astype(jnp.bfloat16), w11_ref[...],
                preferred_element_type=jnp.float32)
    h = jnp.maximum(h * s11_ref[...] + b11_ref[...], 0.0)
    z = h - jnp.max(h, axis=1, keepdims=True)
    e = jnp.exp(z)
    o_ref[...] = e / jnp.sum(e, axis=1, keepdims=True)


def _head(v, head_params):
    B, C = v.shape
    (w9, s9, b9), (w10, s10, b10), (w11, s11, b11) = head_params
    nc = w11.shape[-1]
    return pl.pallas_call(
        _head_body,
        out_shape=jax.ShapeDtypeStruct((B, nc), jnp.float32),
        in_specs=[pl.BlockSpec(v.shape, lambda: (0, 0))] +
                 [pl.BlockSpec(a.shape, lambda: (0, 0))
                  for a in (w9, s9, b9, w10, s10, b10, w11, s11, b11)],
        out_specs=pl.BlockSpec((B, nc), lambda: (0, 0)),
        compiler_params=pltpu.CompilerParams(
            vmem_limit_bytes=_VMEM_LIMIT),
    )(v, w9, s9, b9, w10, s10, b10, w11, s11, b11)


def _pool_pad_t(h_t, D):
    B, C = h_t.shape[0], h_t.shape[1]
    Dp = D + 2
    Sp = Dp ** 3
    P = Dp * Dp + Dp + 1
    v = h_t[:, :, P:P + Sp].reshape(B, C, Dp, Dp, Dp)
    v = v[:, :, 1:1 + D, 1:1 + D, 1:1 + D]
    Do = D // 2
    v = v.reshape(B, C, Do, 2, Do, 2, Do, 2).max(axis=(3, 5, 7))
    Dq = Do + 2
    Pq = Dq * Dq + Dq + 1
    vp = jnp.pad(v, ((0, 0), (0, 0), (1, 1), (1, 1), (1, 1)))
    return jnp.pad(vp.reshape(B, C, Dq ** 3), ((0, 0), (0, 0), (Pq, Pq)))


@jax.jit
def _forward(x, params, head_params):
    B, D = x.shape[0], x.shape[1]
    xb = x.astype(jnp.bfloat16)
    Dp = D + 2
    Sp = Dp ** 3
    xp2 = jnp.pad(xb, ((0, 0), (2, 2), (2, 2), (2, 2)))
    cols = [xp2[:, kd:kd + Dp, kh:kh + Dp, kw:kw + Dp].reshape(B, Sp)
            for kd in range(3) for kh in range(3) for kw in range(3)]
    x27 = jnp.stack(cols, axis=1)

    def tp(prm):
        w, s, b = prm
        return w.T, s.T, b.T

    w1t = params[0][0].T
    w2t, s2t, b2t = tp(params[1])
    h = _conv12_t(x27, w1t, w2t, s2t, b2t, D)
    h = _conv_pair_t(_pool_pad_t(h, D), tp(params[2]), tp(params[3]),
                     D // 2, nb=math.gcd(2, B))
    h = _conv_pair_t(_pool_pad_t(h, D // 2), tp(params[4]), tp(params[5]),
                     D // 4, nb=math.gcd(8, B))
    h = _conv_pair_t(_pool_pad_t(h, D // 4), tp(params[6]), tp(params[7]),
                     D // 8, nb=math.gcd(16, B))
    Dp4 = D // 8 + 2
    P4 = Dp4 * Dp4 + Dp4 + 1
    Sp4 = Dp4 ** 3
    v = h[:, :, P4:P4 + Sp4].reshape(B, 256, Dp4, Dp4, Dp4)
    v = v[:, :, 1:1 + D // 8, 1:1 + D // 8, 1:1 + D // 8]
    v = v.reshape(B, 256, (D // 8) ** 3).max(axis=2)
    return _head(v, head_params)


def kernel(x, w0, s0, sh0, w1, s1, sh1, w2, s2, sh2, w3, s3, sh3,
           w4, s4, sh4, w5, s5, sh5, w6, s6, sh6, w7, s7, sh7,
           w8, s8, sh8, w9, s9, sh9, w10, s10, sh10,
           hw0, hs0, hb0, hw1, hs1, hb1, hw2, hs2, hb2):
    params = [(w0, s0, sh0), (w1, s1, sh1), (w2, s2, sh2), (w3, s3, sh3),
              (w4, s4, sh4), (w5, s5, sh5), (w6, s6, sh6), (w7, s7, sh7)]
    head_params = ((hw0, hs0, hb0), (hw1, hs1, hb1), (hw2, hs2, hb2))
    return _forward(x, params, head_params)


# speedup vs baseline: 3.2486x; 1.2810x over previous
import math

import numpy as np
import jax
import jax.numpy as jnp
from jax.experimental import pallas as pl
from jax.experimental.pallas import tpu as pltpu


_VMEM_LIMIT = 48 * 1024 * 1024


def _tap_offsets(dp, p):
    return tuple(p + (kd - 1) * dp * dp + (kh - 1) * dp + (kw - 1)
                 for kd in range(3) for kh in range(3) for kw in range(3))


def _halo_mask_t(dp):
    m = np.zeros((dp, dp, dp), np.float32)
    m[1:-1, 1:-1, 1:-1] = 1.0
    return jnp.asarray(m.reshape(1, dp * dp * dp))


def _pick_chunk(sp, cap=2048):
    if sp <= cap:
        return sp
    for n in range(2, sp + 1):
        if sp % n == 0 and sp // n <= cap:
            return sp // n
    return sp


def _dot_ta(w, im):
    return jax.lax.dot_general(w, im, (((0,), (0,)), ((), ())),
                               preferred_element_type=jnp.float32)


def _tconv(srcs, w_ref, s_ref, b_ref, mask_ref, dsts, im_refs,
           *, sp, p, dpp, offsets, chunk, cin, relu):
    nb = len(srcs)
    cout = dsts[0].shape[0]
    zlead = jnp.zeros((cout, p + dpp), dsts[0].dtype)
    for dst in dsts:
        dst[:, 0:p + dpp] = zlead
        dst[:, p + sp - dpp:p + sp + p] = zlead
    for ci, base in enumerate(range(dpp, sp - dpp, chunk)):
        im_ref = im_refs[ci % len(im_refs)]
        for e, src in enumerate(srcs):
            for t, off in enumerate(offsets):
                im_ref[t * cin:(t + 1) * cin, e * chunk:(e + 1) * chunk] = \
                    src[:, off + base:off + base + chunk]
        acc = _dot_ta(w_ref[...], im_ref[...])
        res = acc * s_ref[...] + b_ref[...]
        if relu:
            res = jnp.maximum(res, 0.0)
        msk = mask_ref[:, base:base + chunk]
        for e, dst in enumerate(dsts):
            r = res[:, e * chunk:(e + 1) * chunk] * msk
            dst[:, p + base:p + base + chunk] = r.astype(dst.dtype)


def _make_c12_body(sp, p, dpp, offsets, chunk, c1_out):
    def _body(x_ref, w1_ref, w2_ref, s2_ref, b2_ref, mask_ref, o_ref,
              mid_ref, im1_ref, im_a_ref, im_b_ref):
        mid_ref[:, 0:p + dpp] = jnp.zeros((c1_out, p + dpp), mid_ref.dtype)
        mid_ref[:, p + sp - dpp:p + sp + p] = \
            jnp.zeros((c1_out, p + dpp), mid_ref.dtype)
        for base in range(dpp, sp - dpp, chunk):
            for t, off in enumerate(offsets):
                im1_ref[t:t + 1, :] = \
                    x_ref[:, off + base:off + base + chunk]
            acc = _dot_ta(w1_ref[...], im1_ref[...])
            nrm = jnp.sqrt(jnp.sum(acc * acc, axis=0, keepdims=True)) + 1e-9
            res = jnp.maximum(acc / nrm, 0.0) * mask_ref[:, base:base + chunk]
            mid_ref[:, p + base:p + base + chunk] = res.astype(mid_ref.dtype)
        _tconv([mid_ref], w2_ref, s2_ref, b2_ref, mask_ref, [o_ref.at[0]],
               [im_a_ref, im_b_ref], sp=sp, p=p, dpp=dpp, offsets=offsets,
               chunk=chunk, cin=c1_out, relu=False)
    return _body


def _conv12_t(x_flat, w1, w2, s2t, b2t, D):
    B = x_flat.shape[0]
    Dp = D + 2
    Sp = Dp ** 3
    P = Dp * Dp + Dp + 1
    Dpp = Dp * Dp
    chunk = _pick_chunk(Sp - 2 * Dpp)
    offsets = _tap_offsets(Dp, P)
    mask = _halo_mask_t(Dp)
    c1_out = w1.shape[-1]
    c2_out = w2.shape[-1]
    return pl.pallas_call(
        _make_c12_body(Sp, P, Dpp, offsets, chunk, c1_out),
        out_shape=jax.ShapeDtypeStruct((B, c2_out, Sp + 2 * P), jnp.bfloat16),
        grid=(B,),
        in_specs=[
            pl.BlockSpec((None, 1, Sp + 2 * P), lambda b: (b, 0, 0)),
            pl.BlockSpec((27, c1_out), lambda b: (0, 0)),
            pl.BlockSpec((27 * c1_out, c2_out), lambda b: (0, 0)),
            pl.BlockSpec((c2_out, 1), lambda b: (0, 0)),
            pl.BlockSpec((c2_out, 1), lambda b: (0, 0)),
            pl.BlockSpec((1, Sp), lambda b: (0, 0)),
        ],
        out_specs=pl.BlockSpec((1, c2_out, Sp + 2 * P), lambda b: (b, 0, 0)),
        scratch_shapes=[
            pltpu.VMEM((c1_out, Sp + 2 * P), jnp.bfloat16),
            pltpu.VMEM((27, chunk), jnp.bfloat16),
            pltpu.VMEM((27 * c1_out, chunk), jnp.bfloat16),
            pltpu.VMEM((27 * c1_out, chunk), jnp.bfloat16),
        ],
        compiler_params=pltpu.CompilerParams(
            dimension_semantics=("parallel",),
            vmem_limit_bytes=_VMEM_LIMIT),
    )(x_flat, w1, w2, s2t, b2t, mask)


def _make_pair_body(sp, p, dpp, offsets, chunk, nb, cin_a, cin_b):
    def _body(x_ref, wa_ref, sa_ref, ba_ref, wb_ref, sb_ref, bb_ref,
              mask_ref, o_ref, mid_ref, im_a_ref, im_b_ref):
        xs = [x_ref.at[e] for e in range(nb)]
        mids = [mid_ref.at[e] for e in range(nb)]
        outs = [o_ref.at[e] for e in range(nb)]
        _tconv(xs, wa_ref, sa_ref, ba_ref, mask_ref, mids, [im_a_ref],
               sp=sp, p=p, dpp=dpp, offsets=offsets, chunk=chunk,
               cin=cin_a, relu=True)
        _tconv(mids, wb_ref, sb_ref, bb_ref, mask_ref, outs, [im_b_ref],
               sp=sp, p=p, dpp=dpp, offsets=offsets, chunk=chunk,
               cin=cin_b, relu=True)
    return _body


def _conv_pair_t(x_t, pa, pb, D, nb):
    B, cin, total = x_t.shape
    Dp = D + 2
    Sp = Dp ** 3
    P = Dp * Dp + Dp + 1
    Dpp = Dp * Dp
    assert total == Sp + 2 * P and B % nb == 0
    wat, sat, bat = pa
    wbt, sbt, bbt = pb
    ca_out = wat.shape[-1]
    cb_out = wbt.shape[-1]
    chunk = Sp - 2 * Dpp
    offsets = _tap_offsets(Dp, P)
    mask = _halo_mask_t(Dp)
    return pl.pallas_call(
        _make_pair_body(Sp, P, Dpp, offsets, chunk, nb, cin, ca_out),
        out_shape=jax.ShapeDtypeStruct((B, cb_out, total), jnp.bfloat16),
        grid=(B // nb,),
        in_specs=[
            pl.BlockSpec((nb, cin, total), lambda b: (b, 0, 0)),
            pl.BlockSpec((27 * cin, ca_out), lambda b: (0, 0)),
            pl.BlockSpec((ca_out, 1), lambda b: (0, 0)),
            pl.BlockSpec((ca_out, 1), lambda b: (0, 0)),
            pl.BlockSpec((27 * ca_out, cb_out), lambda b: (0, 0)),
            pl.BlockSpec((cb_out, 1), lambda b: (0, 0)),
            pl.BlockSpec((cb_out, 1), lambda b: (0, 0)),
            pl.BlockSpec((1, Sp), lambda b: (0, 0)),
        ],
        out_specs=pl.BlockSpec((nb, cb_out, total), lambda b: (b, 0, 0)),
        scratch_shapes=[
            pltpu.VMEM((nb, ca_out, total), jnp.bfloat16),
            pltpu.VMEM((27 * cin, nb * chunk), jnp.bfloat16),
            pltpu.VMEM((27 * ca_out, nb * chunk), jnp.bfloat16),
        ],
        compiler_params=pltpu.CompilerParams(
            dimension_semantics=("parallel",),
            vmem_limit_bytes=_VMEM_LIMIT),
    )(x_t, wat, sat, bat, wbt, sbt, bbt, mask)


def _head_body(x_ref, w9_ref, s9_ref, b9_ref, w10_ref, s10_ref, b10_ref,
               w11_ref, s11_ref, b11_ref, o_ref):
    h = jnp.dot(x_ref[...], w9_ref[...], preferred_element_type=jnp.float32)
    h = jnp.maximum(h * s9_ref[...] + b9_ref[...], 0.0)
    h = jnp.dot(h.astype(jnp.bfloat16), w10_ref[...],
                preferred_element_type=jnp.float32)
    h = jnp.maximum(h * s10_ref[...] + b10_ref[...], 0.0)
    h = jnp.dot(h.astype(jnp.bfloat16), w11_ref[...],
                preferred_element_type=jnp.float32)
    h = jnp.maximum(h * s11_ref[...] + b11_ref[...], 0.0)
    z = h - jnp.max(h, axis=1, keepdims=True)
    e = jnp.exp(z)
    o_ref[...] = e / jnp.sum(e, axis=1, keepdims=True)


def _head(v, head_params):
    B, C = v.shape
    (w9, s9, b9), (w10, s10, b10), (w11, s11, b11) = head_params
    nc = w11.shape[-1]
    return pl.pallas_call(
        _head_body,
        out_shape=jax.ShapeDtypeStruct((B, nc), jnp.float32),
        in_specs=[pl.BlockSpec(v.shape, lambda: (0, 0))] +
                 [pl.BlockSpec(a.shape, lambda: (0, 0))
                  for a in (w9, s9, b9, w10, s10, b10, w11, s11, b11)],
        out_specs=pl.BlockSpec((B, nc), lambda: (0, 0)),
        compiler_params=pltpu.CompilerParams(
            vmem_limit_bytes=_VMEM_LIMIT),
    )(v, w9, s9, b9, w10, s10, b10, w11, s11, b11)


def _pool_pad_t(h_t, D):
    B, C = h_t.shape[0], h_t.shape[1]
    Dp = D + 2
    Sp = Dp ** 3
    P = Dp * Dp + Dp + 1
    v = h_t[:, :, P:P + Sp].reshape(B, C, Dp, Dp, Dp)
    v = v[:, :, 1:1 + D, 1:1 + D, 1:1 + D]
    Do = D // 2
    v = v.reshape(B, C, Do, 2, Do, 2, Do, 2).max(axis=(3, 5, 7))
    Dq = Do + 2
    Pq = Dq * Dq + Dq + 1
    vp = jnp.pad(v, ((0, 0), (0, 0), (1, 1), (1, 1), (1, 1)))
    return jnp.pad(vp.reshape(B, C, Dq ** 3), ((0, 0), (0, 0), (Pq, Pq)))


@jax.jit
def _forward(x, params, head_params):
    B, D = x.shape[0], x.shape[1]
    xb = x.astype(jnp.bfloat16)
    Dp = D + 2
    Sp = Dp ** 3
    P = Dp * Dp + Dp + 1
    xp = jnp.pad(xb, ((0, 0), (1, 1), (1, 1), (1, 1)))
    x_flat = jnp.pad(xp.reshape(B, 1, Sp), ((0, 0), (0, 0), (P, P)))

    def tp(prm):
        w, s, b = prm
        return w, s.T, b.T

    h = _conv12_t(x_flat, params[0][0], params[1][0],
                  params[1][1].T, params[1][2].T, D)
    h = _conv_pair_t(_pool_pad_t(h, D), tp(params[2]), tp(params[3]),
                     D // 2, nb=math.gcd(2, B))
    h = _conv_pair_t(_pool_pad_t(h, D // 2), tp(params[4]), tp(params[5]),
                     D // 4, nb=math.gcd(8, B))
    h = _conv_pair_t(_pool_pad_t(h, D // 4), tp(params[6]), tp(params[7]),
                     D // 8, nb=math.gcd(16, B))
    Dp4 = D // 8 + 2
    P4 = Dp4 * Dp4 + Dp4 + 1
    Sp4 = Dp4 ** 3
    v = h[:, :, P4:P4 + Sp4].reshape(B, 256, Dp4, Dp4, Dp4)
    v = v[:, :, 1:1 + D // 8, 1:1 + D // 8, 1:1 + D // 8]
    v = v.reshape(B, 256, (D // 8) ** 3).max(axis=2)
    return _head(v, head_params)


def kernel(x, w0, s0, sh0, w1, s1, sh1, w2, s2, sh2, w3, s3, sh3,
           w4, s4, sh4, w5, s5, sh5, w6, s6, sh6, w7, s7, sh7,
           w8, s8, sh8, w9, s9, sh9, w10, s10, sh10,
           hw0, hs0, hb0, hw1, hs1, hb1, hw2, hs2, hb2):
    params = [(w0, s0, sh0), (w1, s1, sh1), (w2, s2, sh2), (w3, s3, sh3),
              (w4, s4, sh4), (w5, s5, sh5), (w6, s6, sh6), (w7, s7, sh7)]
    head_params = ((hw0, hs0, hb0), (hw1, hs1, hb1), (hw2, hs2, hb2))
    return _forward(x, params, head_params)
